# baseline (speedup 1.0000x reference)
import functools

import jax
import jax.numpy as jnp
from jax import lax
from jax.experimental import pallas as pl
from jax.experimental.pallas import tpu as pltpu
from jax.experimental.pallas import tpu_sc as plsc

NC = 2
NS = 16
NW = NC * NS
LANES = 16

VOCAB = 1_000_000
EMBED_DIM = 32
BATCH = 16384
HIST = 200
BT = BATCH // 128
GRP = 4
BLK_B = GRP * 128
BLOCKS = HIST * BT // GRP
N_BLK = BLOCKS // NW
JPH = BT // GRP


def _transpose_block(rows, blk):
    iota = lax.iota(jnp.int32, LANES)

    @pl.loop(0, EMBED_DIM)
    def _col(e):
        et = e // 8
        er = e % 8
        cidx = jnp.full((LANES,), e, jnp.int32)
        for g in range(GRP):
            for c0 in range(0, 128, LANES):
                ridx = iota + (g * 128 + c0)
                v = plsc.load_gather(rows, [ridx, cidx])
                blk[et, g, er, pl.ds(c0, LANES)] = v


def _gather_body(table_hbm, idx_hbm, out_hbm, idx_v, rows_v, blk_v, *sems):
    sem_i = sems[0:2]
    sem_g = sems[2:4]
    sem_w = sems[4:6]
    wid = lax.axis_index("s") * NC + lax.axis_index("c")
    wbase = wid * N_BLK

    def idx_src(i):
        return idx_hbm.at[pl.ds((wbase + i) * BLK_B, BLK_B)]

    def out_dst(i):
        gi = wbase + i
        return out_hbm.at[gi // JPH, :, pl.ds((gi % JPH) * GRP, GRP)]

    def step(i, k, *, first, last, prefetch=True):
        pltpu.make_async_copy(table_hbm.at[idx_v.at[k]], rows_v.at[k], sem_g[k]).wait()
        if prefetch:
            pltpu.async_copy(idx_src(i + 2), idx_v.at[k], sem_i[k])
        if not last:
            pltpu.make_async_copy(idx_src(i + 1), idx_v.at[1 - k], sem_i[1 - k]).wait()
            pltpu.async_copy(
                table_hbm.at[idx_v.at[1 - k]], rows_v.at[1 - k], sem_g[1 - k]
            )
        if not first:
            pltpu.make_async_copy(blk_v.at[k], out_dst(i - 2), sem_w[k]).wait()
        _transpose_block(rows_v.at[k], blk_v.at[k])
        pltpu.async_copy(blk_v.at[k], out_dst(i), sem_w[k])

    pltpu.async_copy(idx_src(0), idx_v.at[0], sem_i[0])
    pltpu.make_async_copy(idx_src(0), idx_v.at[0], sem_i[0]).wait()
    pltpu.async_copy(idx_src(1), idx_v.at[1], sem_i[1])
    pltpu.async_copy(table_hbm.at[idx_v.at[0]], rows_v.at[0], sem_g[0])

    step(0, 0, first=True, last=False)
    step(1, 1, first=True, last=False)

    @pl.loop(0, (N_BLK - 4) // 2)
    def _pair(h):
        i = 2 + h * 2
        step(i, 0, first=False, last=False)
        step(i + 1, 1, first=False, last=False)

    step(N_BLK - 2, 0, first=False, last=False, prefetch=False)
    step(N_BLK - 1, 1, first=False, last=True, prefetch=False)

    pltpu.make_async_copy(blk_v.at[0], out_dst(N_BLK - 2), sem_w[0]).wait()
    pltpu.make_async_copy(blk_v.at[1], out_dst(N_BLK - 1), sem_w[1]).wait()


_gather = functools.partial(
    pl.kernel,
    out_type=jax.ShapeDtypeStruct((HIST, 4, BT, 8, 128), jnp.float32),
    mesh=plsc.VectorSubcoreMesh(
        core_axis_name="c", subcore_axis_name="s", num_cores=NC, num_subcores=NS
    ),
    scratch_types=[
        pltpu.VMEM((2, BLK_B), jnp.int32),
        pltpu.VMEM((2, BLK_B, EMBED_DIM), jnp.float32),
        pltpu.VMEM((2, 4, GRP, 8, 128), jnp.float32),
        pltpu.SemaphoreType.DMA,
        pltpu.SemaphoreType.DMA,
        pltpu.SemaphoreType.DMA,
        pltpu.SemaphoreType.DMA,
        pltpu.SemaphoreType.DMA,
        pltpu.SemaphoreType.DMA,
    ],
    compiler_params=pltpu.CompilerParams(
        use_tc_tiling_on_sc=False, needs_layout_passes=False
    ),
)(_gather_body)


@jax.jit
def kernel(input_ids, table):
    ids = input_ids.T.reshape(-1).astype(jnp.int32)
    x = _gather(table, ids)
    x = x.transpose(2, 4, 0, 1, 3)
    return x.reshape(BATCH, HIST, EMBED_DIM)


# speedup vs baseline: 1.5020x; 1.5020x over previous
import functools

import jax
import jax.numpy as jnp
from jax import lax
from jax.experimental import pallas as pl
from jax.experimental.pallas import tpu as pltpu
from jax.experimental.pallas import tpu_sc as plsc

NC = 2
NS = 16
NW = NC * NS
LANES = 16

VOCAB = 1_000_000
EMBED_DIM = 32
BATCH = 16384
HIST = 200
BT = BATCH // 128
GRP = 4
BLK_B = GRP * 128
BLOCKS = HIST * BT // GRP
N_BLK = BLOCKS // NW
JPH = BT // GRP


def _transpose_block(rows, blk, iota, cidx):
    @pl.loop(0, GRP * (128 // LANES))
    def _grp(j):
        g = j // (128 // LANES)
        c0 = (j % (128 // LANES)) * LANES
        ridx = iota + j * LANES
        for e0 in range(0, EMBED_DIM, 8):
            vs = [plsc.load_gather(rows, [ridx, cidx[e0 + t]]) for t in range(8)]
            for t in range(8):
                e = e0 + t
                blk[e // 8, g, e % 8, pl.ds(c0, LANES)] = vs[t]


def _gather_body(table_hbm, idx_hbm, out_hbm, idx_v, rows_v, blk_v, *sems):
    sem_i = sems[0:2]
    sem_g = sems[2:4]
    sem_w = sems[4:6]
    wid = lax.axis_index("s") * NC + lax.axis_index("c")
    wbase = wid * N_BLK
    iota = lax.iota(jnp.int32, LANES)
    cidx = [jnp.full((LANES,), e, jnp.int32) for e in range(EMBED_DIM)]

    def idx_src(i):
        return idx_hbm.at[pl.ds((wbase + i) * BLK_B, BLK_B)]

    def out_dst(i):
        gi = wbase + i
        return out_hbm.at[gi // JPH, :, pl.ds((gi % JPH) * GRP, GRP)]

    def step(i, k, *, first, last, prefetch=True):
        pltpu.make_async_copy(table_hbm.at[idx_v.at[k]], rows_v.at[k], sem_g[k]).wait()
        if prefetch:
            pltpu.async_copy(idx_src(i + 2), idx_v.at[k], sem_i[k])
        if not last:
            pltpu.make_async_copy(idx_src(i + 1), idx_v.at[1 - k], sem_i[1 - k]).wait()
            pltpu.async_copy(
                table_hbm.at[idx_v.at[1 - k]], rows_v.at[1 - k], sem_g[1 - k]
            )
        if not first:
            pltpu.make_async_copy(blk_v.at[k], out_dst(i - 2), sem_w[k]).wait()
        _transpose_block(rows_v.at[k], blk_v.at[k], iota, cidx)
        pltpu.async_copy(blk_v.at[k], out_dst(i), sem_w[k])

    pltpu.async_copy(idx_src(0), idx_v.at[0], sem_i[0])
    pltpu.make_async_copy(idx_src(0), idx_v.at[0], sem_i[0]).wait()
    pltpu.async_copy(idx_src(1), idx_v.at[1], sem_i[1])
    pltpu.async_copy(table_hbm.at[idx_v.at[0]], rows_v.at[0], sem_g[0])

    step(0, 0, first=True, last=False)
    step(1, 1, first=True, last=False)

    @pl.loop(0, (N_BLK - 4) // 2)
    def _pair(h):
        i = 2 + h * 2
        step(i, 0, first=False, last=False)
        step(i + 1, 1, first=False, last=False)

    step(N_BLK - 2, 0, first=False, last=False, prefetch=False)
    step(N_BLK - 1, 1, first=False, last=True, prefetch=False)

    pltpu.make_async_copy(blk_v.at[0], out_dst(N_BLK - 2), sem_w[0]).wait()
    pltpu.make_async_copy(blk_v.at[1], out_dst(N_BLK - 1), sem_w[1]).wait()


_gather = functools.partial(
    pl.kernel,
    out_type=jax.ShapeDtypeStruct((HIST, 4, BT, 8, 128), jnp.float32),
    mesh=plsc.VectorSubcoreMesh(
        core_axis_name="c", subcore_axis_name="s", num_cores=NC, num_subcores=NS
    ),
    scratch_types=[
        pltpu.VMEM((2, BLK_B), jnp.int32),
        pltpu.VMEM((2, BLK_B, EMBED_DIM), jnp.float32),
        pltpu.VMEM((2, 4, GRP, 8, 128), jnp.float32),
        pltpu.SemaphoreType.DMA,
        pltpu.SemaphoreType.DMA,
        pltpu.SemaphoreType.DMA,
        pltpu.SemaphoreType.DMA,
        pltpu.SemaphoreType.DMA,
        pltpu.SemaphoreType.DMA,
    ],
    compiler_params=pltpu.CompilerParams(
        use_tc_tiling_on_sc=False, needs_layout_passes=False
    ),
)(_gather_body)


@jax.jit
def kernel(input_ids, table):
    ids = input_ids.T.reshape(-1).astype(jnp.int32)
    x = _gather(table, ids)
    x = x.transpose(2, 4, 0, 1, 3)
    return x.reshape(BATCH, HIST, EMBED_DIM)


# speedup vs baseline: 2.7980x; 1.8629x over previous
import functools

import jax
import jax.numpy as jnp
from jax import lax
from jax.experimental import pallas as pl
from jax.experimental.pallas import tpu as pltpu
from jax.experimental.pallas import tpu_sc as plsc

NC = 2
NS = 16
NW = NC * NS
LANES = 16

VOCAB = 1_000_000
EMBED_DIM = 32
BATCH = 16384
HIST = 200
BT = BATCH // 128
GRP = 4
BLK_B = GRP * 128
BLOCKS = HIST * BT // GRP
N_BLK = BLOCKS // NW
JPH = BT // GRP


UNROLL_R = 4


def _transpose_block(rows, blk, et_v, er_v):
    @pl.loop(0, BLK_B // UNROLL_R)
    def _rows(j):
        r0 = j * UNROLL_R
        for u in range(UNROLL_R):
            r = r0 + u
            g_v = jnp.full((LANES,), r // 128, jnp.int32)
            bc_v = jnp.full((LANES,), r % 128, jnp.int32)
            v0 = rows[r, pl.ds(0, LANES)]
            v1 = rows[r, pl.ds(LANES, LANES)]
            plsc.store_scatter(blk, [et_v[0], g_v, er_v, bc_v], v0)
            plsc.store_scatter(blk, [et_v[1], g_v, er_v, bc_v], v1)


def _gather_body(table_hbm, idx_hbm, out_hbm, idx_v, rows_v, blk_v, *sems):
    sem_i = sems[0:2]
    sem_g = sems[2:4]
    sem_w = sems[4:6]
    wid = lax.axis_index("s") * NC + lax.axis_index("c")
    wbase = wid * N_BLK
    iota = lax.iota(jnp.int32, LANES)
    et_v = (iota // 8, iota // 8 + 2)
    er_v = iota % 8

    def idx_src(i):
        return idx_hbm.at[pl.ds((wbase + i) * BLK_B, BLK_B)]

    def out_dst(i):
        gi = wbase + i
        return out_hbm.at[gi // JPH, :, pl.ds((gi % JPH) * GRP, GRP)]

    def blk_src(k):
        return blk_v.at[k, :, :, :, pl.ds(0, 128)]

    def step(i, k, *, first, last, prefetch=True):
        pltpu.make_async_copy(table_hbm.at[idx_v.at[k]], rows_v.at[k], sem_g[k]).wait()
        if prefetch:
            pltpu.async_copy(idx_src(i + 2), idx_v.at[k], sem_i[k])
        if not last:
            pltpu.make_async_copy(idx_src(i + 1), idx_v.at[1 - k], sem_i[1 - k]).wait()
            pltpu.async_copy(
                table_hbm.at[idx_v.at[1 - k]], rows_v.at[1 - k], sem_g[1 - k]
            )
        if not first:
            pltpu.make_async_copy(blk_src(k), out_dst(i - 2), sem_w[k]).wait()
        _transpose_block(rows_v.at[k], blk_v.at[k], et_v, er_v)
        pltpu.async_copy(blk_src(k), out_dst(i), sem_w[k])

    pltpu.async_copy(idx_src(0), idx_v.at[0], sem_i[0])
    pltpu.make_async_copy(idx_src(0), idx_v.at[0], sem_i[0]).wait()
    pltpu.async_copy(idx_src(1), idx_v.at[1], sem_i[1])
    pltpu.async_copy(table_hbm.at[idx_v.at[0]], rows_v.at[0], sem_g[0])

    step(0, 0, first=True, last=False)
    step(1, 1, first=True, last=False)

    @pl.loop(0, (N_BLK - 4) // 2)
    def _pair(h):
        i = 2 + h * 2
        step(i, 0, first=False, last=False)
        step(i + 1, 1, first=False, last=False)

    step(N_BLK - 2, 0, first=False, last=False, prefetch=False)
    step(N_BLK - 1, 1, first=False, last=True, prefetch=False)

    pltpu.make_async_copy(blk_src(0), out_dst(N_BLK - 2), sem_w[0]).wait()
    pltpu.make_async_copy(blk_src(1), out_dst(N_BLK - 1), sem_w[1]).wait()


_gather = functools.partial(
    pl.kernel,
    out_type=jax.ShapeDtypeStruct((HIST, 4, BT, 8, 128), jnp.float32),
    mesh=plsc.VectorSubcoreMesh(
        core_axis_name="c", subcore_axis_name="s", num_cores=NC, num_subcores=NS
    ),
    scratch_types=[
        pltpu.VMEM((2, BLK_B), jnp.int32),
        pltpu.VMEM((2, BLK_B, EMBED_DIM), jnp.float32),
        pltpu.VMEM((2, 4, GRP, 8, 129), jnp.float32),
        pltpu.SemaphoreType.DMA,
        pltpu.SemaphoreType.DMA,
        pltpu.SemaphoreType.DMA,
        pltpu.SemaphoreType.DMA,
        pltpu.SemaphoreType.DMA,
        pltpu.SemaphoreType.DMA,
    ],
    compiler_params=pltpu.CompilerParams(
        use_tc_tiling_on_sc=False, needs_layout_passes=False
    ),
)(_gather_body)


@jax.jit
def kernel(input_ids, table):
    ids = input_ids.T.reshape(-1).astype(jnp.int32)
    x = _gather(table, ids)
    x = x.transpose(2, 4, 0, 1, 3)
    return x.reshape(BATCH, HIST, EMBED_DIM)


# speedup vs baseline: 2.9184x; 1.0430x over previous
import functools

import jax
import jax.numpy as jnp
from jax import lax
from jax.experimental import pallas as pl
from jax.experimental.pallas import tpu as pltpu
from jax.experimental.pallas import tpu_sc as plsc

NC = 2
NS = 16
NW = NC * NS
LANES = 16

VOCAB = 1_000_000
EMBED_DIM = 32
BATCH = 16384
HIST = 200
BT = BATCH // 128
GRP = 4
BLK_B = GRP * 128
BLOCKS = HIST * BT // GRP
N_BLK = BLOCKS // NW
JPH = BT // GRP


UNROLL_R = 8


def _transpose_block(rows, blk, et_v, er_v):
    @pl.loop(0, BLK_B // UNROLL_R)
    def _rows(j):
        r0 = j * UNROLL_R
        for u in range(UNROLL_R):
            r = r0 + u
            g_v = jnp.full((LANES,), r // 128, jnp.int32)
            bc_v = jnp.full((LANES,), r % 128, jnp.int32)
            v0 = rows[r, pl.ds(0, LANES)]
            v1 = rows[r, pl.ds(LANES, LANES)]
            plsc.store_scatter(blk, [et_v[0], g_v, er_v, bc_v], v0)
            plsc.store_scatter(blk, [et_v[1], g_v, er_v, bc_v], v1)


def _gather_body(table_hbm, idx_hbm, out_hbm, idx_v, rows_v, blk_v, *sems):
    sem_i = sems[0:2]
    sem_g = sems[2:4]
    sem_w = sems[4:6]
    wid = lax.axis_index("s") * NC + lax.axis_index("c")
    wbase = wid * N_BLK
    iota = lax.iota(jnp.int32, LANES)
    et_v = (iota // 8, iota // 8 + 2)
    er_v = iota % 8

    def idx_src(i):
        return idx_hbm.at[pl.ds((wbase + i) * BLK_B, BLK_B)]

    def out_dst(i):
        gi = wbase + i
        return out_hbm.at[gi // JPH, :, pl.ds((gi % JPH) * GRP, GRP)]

    def blk_src(k):
        return blk_v.at[k, :, :, pl.ds(0, 8), pl.ds(0, 128)]

    def step(i, k, *, first, last, prefetch=True):
        pltpu.make_async_copy(table_hbm.at[idx_v.at[k]], rows_v.at[k], sem_g[k]).wait()
        if prefetch:
            pltpu.async_copy(idx_src(i + 2), idx_v.at[k], sem_i[k])
        if not last:
            pltpu.make_async_copy(idx_src(i + 1), idx_v.at[1 - k], sem_i[1 - k]).wait()
            pltpu.async_copy(
                table_hbm.at[idx_v.at[1 - k]], rows_v.at[1 - k], sem_g[1 - k]
            )
        if not first:
            pltpu.make_async_copy(blk_src(k), out_dst(i - 2), sem_w[k]).wait()
        _transpose_block(rows_v.at[k], blk_v.at[k], et_v, er_v)
        pltpu.async_copy(blk_src(k), out_dst(i), sem_w[k])

    pltpu.async_copy(idx_src(0), idx_v.at[0], sem_i[0])
    pltpu.make_async_copy(idx_src(0), idx_v.at[0], sem_i[0]).wait()
    pltpu.async_copy(idx_src(1), idx_v.at[1], sem_i[1])
    pltpu.async_copy(table_hbm.at[idx_v.at[0]], rows_v.at[0], sem_g[0])

    step(0, 0, first=True, last=False)
    step(1, 1, first=True, last=False)

    @pl.loop(0, (N_BLK - 4) // 2)
    def _pair(h):
        i = 2 + h * 2
        step(i, 0, first=False, last=False)
        step(i + 1, 1, first=False, last=False)

    step(N_BLK - 2, 0, first=False, last=False, prefetch=False)
    step(N_BLK - 1, 1, first=False, last=True, prefetch=False)

    pltpu.make_async_copy(blk_src(0), out_dst(N_BLK - 2), sem_w[0]).wait()
    pltpu.make_async_copy(blk_src(1), out_dst(N_BLK - 1), sem_w[1]).wait()


_gather = functools.partial(
    pl.kernel,
    out_type=jax.ShapeDtypeStruct((HIST, 4, BT, 8, 128), jnp.float32),
    mesh=plsc.VectorSubcoreMesh(
        core_axis_name="c", subcore_axis_name="s", num_cores=NC, num_subcores=NS
    ),
    scratch_types=[
        pltpu.VMEM((2, BLK_B), jnp.int32),
        pltpu.VMEM((2, BLK_B, EMBED_DIM), jnp.float32),
        pltpu.VMEM((2, 4, GRP, 10, 129), jnp.float32),
        pltpu.SemaphoreType.DMA,
        pltpu.SemaphoreType.DMA,
        pltpu.SemaphoreType.DMA,
        pltpu.SemaphoreType.DMA,
        pltpu.SemaphoreType.DMA,
        pltpu.SemaphoreType.DMA,
    ],
    compiler_params=pltpu.CompilerParams(
        use_tc_tiling_on_sc=False, needs_layout_passes=False
    ),
)(_gather_body)


@jax.jit
def kernel(input_ids, table):
    ids = input_ids.T.reshape(-1).astype(jnp.int32)
    x = _gather(table, ids)
    x = x.transpose(2, 4, 0, 1, 3)
    return x.reshape(BATCH, HIST, EMBED_DIM)
